# Initial kernel scaffold; baseline (speedup 1.0000x reference)
import jax
import jax.numpy as jnp
from jax.experimental import pallas as pl


def kernel(x, edge_index, W1, b1, W2, b2, fc_W, fc_b):
    raise NotImplementedError("write your pallas kernel here")


# speedup vs baseline: 20.0019x; 20.0019x over previous
import functools

import jax
import jax.numpy as jnp
from jax import lax
from jax.experimental import pallas as pl
from jax.experimental.pallas import tpu as pltpu
from jax.experimental.pallas import tpu_sc as plsc

N = 10000
E = 320000
D = 128
NROWS = 10240
PAD_ROWS = NROWS - N

NC = 2
NS = 16
NW = NC * NS
B = 128
CPT = -(-E // (NW * B))
E_PAD = CPT * NW * B
RPT = NROWS // NS

_MESH = plsc.VectorSubcoreMesh(core_axis_name="c", subcore_axis_name="s")


def _worker_id():
    return lax.axis_index("s") * NC + lax.axis_index("c")


def _make_edge_pass(d_feat):
    @functools.partial(
        pl.kernel,
        mesh=_MESH,
        out_type=jax.ShapeDtypeStruct((NC * NROWS, d_feat), jnp.float32),
        scratch_types=[
            pltpu.VMEM((B,), jnp.int32),
            pltpu.VMEM((B,), jnp.int32),
            pltpu.VMEM((B, d_feat), jnp.float32),
            pltpu.VMEM_SHARED((NROWS, d_feat), jnp.float32),
            pltpu.SemaphoreType.DMA,
        ],
    )
    def edge_pass(table_hbm, src_hbm, dst_hbm, zeros_hbm, out_hbm,
                  src_v, dst_v, vals_v, acc_sh, sem):
        c = lax.axis_index("c")
        s = lax.axis_index("s")
        wid = _worker_id()
        r0 = s * RPT
        pltpu.sync_copy(zeros_hbm.at[pl.ds(r0, RPT)], acc_sh.at[pl.ds(r0, RPT)])
        plsc.subcore_barrier()

        ebase = wid * (CPT * B)

        def body(j, carry):
            base = ebase + j * B
            pltpu.sync_copy(src_hbm.at[pl.ds(base, B)], src_v)
            pltpu.sync_copy(dst_hbm.at[pl.ds(base, B)], dst_v)
            pltpu.async_copy(table_hbm.at[src_v], vals_v, sem).wait()
            pltpu.sync_copy(vals_v, acc_sh.at[dst_v], add=True)
            return carry

        lax.fori_loop(0, CPT, body, 0)
        plsc.subcore_barrier()
        pltpu.sync_copy(acc_sh.at[pl.ds(r0, RPT)],
                        out_hbm.at[pl.ds(c * NROWS + r0, RPT)])

    return edge_pass


def _make_scalar_edge_pass():
    @functools.partial(
        pl.kernel,
        mesh=_MESH,
        out_type=jax.ShapeDtypeStruct((NC * NROWS,), jnp.float32),
        scratch_types=[
            pltpu.VMEM((B,), jnp.int32),
            pltpu.VMEM((B,), jnp.int32),
            pltpu.VMEM((B,), jnp.float32),
            pltpu.VMEM_SHARED((NROWS,), jnp.float32),
            pltpu.VMEM_SHARED((NROWS,), jnp.float32),
            pltpu.SemaphoreType.DMA,
        ],
    )
    def scalar_pass(table_hbm, src_hbm, dst_hbm, zeros_hbm, out_hbm,
                    src_v, dst_v, vals_v, tab_sh, acc_sh, sem):
        c = lax.axis_index("c")
        s = lax.axis_index("s")
        wid = _worker_id()
        r0 = s * RPT
        pltpu.sync_copy(zeros_hbm.at[pl.ds(r0, RPT)], acc_sh.at[pl.ds(r0, RPT)])
        pltpu.sync_copy(table_hbm.at[pl.ds(r0, RPT)], tab_sh.at[pl.ds(r0, RPT)])
        plsc.subcore_barrier()

        ebase = wid * (CPT * B)

        def body(j, carry):
            base = ebase + j * B
            pltpu.sync_copy(src_hbm.at[pl.ds(base, B)], src_v)
            pltpu.sync_copy(dst_hbm.at[pl.ds(base, B)], dst_v)
            pltpu.async_copy(tab_sh.at[src_v], vals_v, sem).wait()
            pltpu.sync_copy(vals_v, acc_sh.at[dst_v], add=True)
            return carry

        lax.fori_loop(0, CPT, body, 0)
        plsc.subcore_barrier()
        pltpu.sync_copy(acc_sh.at[pl.ds(r0, RPT)],
                        out_hbm.at[pl.ds(c * NROWS + r0, RPT)])

    return scalar_pass


def _make_deg_pass():
    @functools.partial(
        pl.kernel,
        mesh=_MESH,
        out_type=jax.ShapeDtypeStruct((NC * NROWS,), jnp.float32),
        scratch_types=[
            pltpu.VMEM((B,), jnp.int32),
            pltpu.VMEM((B,), jnp.float32),
            pltpu.VMEM_SHARED((NROWS,), jnp.float32),
        ],
    )
    def deg_pass(dst_hbm, ones_hbm, zeros_hbm, out_hbm, dst_v, ones_v, acc_sh):
        c = lax.axis_index("c")
        s = lax.axis_index("s")
        wid = _worker_id()
        r0 = s * RPT
        pltpu.sync_copy(zeros_hbm.at[pl.ds(r0, RPT)], acc_sh.at[pl.ds(r0, RPT)])
        pltpu.sync_copy(ones_hbm, ones_v)
        plsc.subcore_barrier()

        ebase = wid * (CPT * B)

        def body(j, carry):
            base = ebase + j * B
            pltpu.sync_copy(dst_hbm.at[pl.ds(base, B)], dst_v)
            pltpu.sync_copy(ones_v, acc_sh.at[dst_v], add=True)
            return carry

        lax.fori_loop(0, CPT, body, 0)
        plsc.subcore_barrier()
        pltpu.sync_copy(acc_sh.at[pl.ds(r0, RPT)],
                        out_hbm.at[pl.ds(c * NROWS + r0, RPT)])

    return deg_pass


_BR = 1024
_GRID = NROWS // _BR


def _tc_matmul_scale(x_pad, W1, degp):
    def body(x_ref, w_ref, degp_ref, xwp_ref, dis_ref):
        deg = degp_ref[0] + degp_ref[1] + 1.0
        dis = lax.rsqrt(deg)
        xw = jnp.dot(x_ref[...], w_ref[...], preferred_element_type=jnp.float32)
        xwp_ref[...] = dis * xw
        dis_ref[...] = dis

    return pl.pallas_call(
        body,
        grid=(_GRID,),
        in_specs=[
            pl.BlockSpec((_BR, D), lambda i: (i, 0)),
            pl.BlockSpec((D, D), lambda i: (0, 0)),
            pl.BlockSpec((NC, _BR, 1), lambda i: (0, i, 0)),
        ],
        out_specs=[
            pl.BlockSpec((_BR, D), lambda i: (i, 0)),
            pl.BlockSpec((_BR, 1), lambda i: (i, 0)),
        ],
        out_shape=[
            jax.ShapeDtypeStruct((NROWS, D), jnp.float32),
            jax.ShapeDtypeStruct((NROWS, 1), jnp.float32),
        ],
    )(x_pad, W1, degp)


def _tc_layer2(accp, xwp, dis, b1, W2, fc_W):
    def body(accp_ref, xwp_ref, dis_ref, b1_ref, w2_ref, fcw_ref, zp_ref):
        i = pl.program_id(0)
        acc = accp_ref[0] + accp_ref[1] + xwp_ref[...]
        h1 = jnp.maximum(dis_ref[...] * acc + b1_ref[...], 0.0)
        w2f = jnp.dot(w2_ref[...], fcw_ref[...], preferred_element_type=jnp.float32)
        z = jnp.dot(h1, w2f, preferred_element_type=jnp.float32)
        rows = i * _BR + lax.broadcasted_iota(jnp.int32, (_BR, 1), 0)
        zp_ref[...] = jnp.where(rows < N, dis_ref[...] * z, 0.0)

    return pl.pallas_call(
        body,
        grid=(_GRID,),
        in_specs=[
            pl.BlockSpec((NC, _BR, D), lambda i: (0, i, 0)),
            pl.BlockSpec((_BR, D), lambda i: (i, 0)),
            pl.BlockSpec((_BR, 1), lambda i: (i, 0)),
            pl.BlockSpec((1, D), lambda i: (0, 0)),
            pl.BlockSpec((D, 64), lambda i: (0, 0)),
            pl.BlockSpec((64, 1), lambda i: (0, 0)),
        ],
        out_specs=pl.BlockSpec((_BR, 1), lambda i: (i, 0)),
        out_shape=jax.ShapeDtypeStruct((NROWS, 1), jnp.float32),
    )(accp, xwp, dis, b1, W2, fc_W)


def _tc_final(acc2p, zp, dis, b2, fc_W, fc_b):
    def body(acc2p_ref, zp_ref, dis_ref, b2_ref, fcw_ref, fcb_ref, out_ref):
        t = dis_ref[...] * (acc2p_ref[0] + acc2p_ref[1] + zp_ref[...])
        bias = jnp.dot(b2_ref[...], fcw_ref[...],
                       preferred_element_type=jnp.float32) + fcb_ref[...]
        out_ref[...] = jax.nn.sigmoid(t + bias)

    return pl.pallas_call(
        body,
        grid=(_GRID,),
        in_specs=[
            pl.BlockSpec((NC, _BR, 1), lambda i: (0, i, 0)),
            pl.BlockSpec((_BR, 1), lambda i: (i, 0)),
            pl.BlockSpec((_BR, 1), lambda i: (i, 0)),
            pl.BlockSpec((1, 64), lambda i: (0, 0)),
            pl.BlockSpec((64, 1), lambda i: (0, 0)),
            pl.BlockSpec((1, 1), lambda i: (0, 0)),
        ],
        out_specs=pl.BlockSpec((_BR, 1), lambda i: (i, 0)),
        out_shape=jax.ShapeDtypeStruct((NROWS, 1), jnp.float32),
    )(acc2p, zp, dis, b2, fc_W, fc_b)


def kernel(x, edge_index, W1, b1, W2, b2, fc_W, fc_b):
    src = edge_index[0].astype(jnp.int32)
    dst = edge_index[1].astype(jnp.int32)
    npad = E_PAD - E
    pad_idx = N + (jnp.arange(npad, dtype=jnp.int32) % PAD_ROWS)
    src_p = jnp.concatenate([src, pad_idx])
    dst_p = jnp.concatenate([dst, pad_idx])

    zeros2d = jnp.zeros((NROWS, D), jnp.float32)
    zeros1d = jnp.zeros((NROWS,), jnp.float32)
    ones_b = jnp.ones((B,), jnp.float32)
    x_pad = jnp.concatenate([x.astype(jnp.float32),
                             jnp.zeros((NROWS - N, D), jnp.float32)])

    degp = _make_deg_pass()(dst_p, ones_b, zeros1d)
    degp = degp.reshape(NC, NROWS, 1)

    xwp, dis = _tc_matmul_scale(x_pad, W1, degp)

    accp = _make_edge_pass(D)(xwp, src_p, dst_p, zeros2d)
    accp = accp.reshape(NC, NROWS, D)

    zp = _tc_layer2(accp, xwp, dis, b1.reshape(1, D), W2, fc_W)

    acc2p = _make_scalar_edge_pass()(zp.reshape(NROWS), src_p, dst_p, zeros1d)
    acc2p = acc2p.reshape(NC, NROWS, 1)

    out = _tc_final(acc2p, zp, dis, b2.reshape(1, 64), fc_W,
                    fc_b.reshape(1, 1))
    return out[:N]


# speedup vs baseline: 26.8109x; 1.3404x over previous
import functools

import jax
import jax.numpy as jnp
from jax import lax
from jax.experimental import pallas as pl
from jax.experimental.pallas import tpu as pltpu
from jax.experimental.pallas import tpu_sc as plsc

N = 10000
E = 320000
D = 128
NROWS = 10240
PAD_ROWS = NROWS - N

NC = 2
NS = 16
NW = NC * NS
B = 128
CPT = 80
CPT2 = CPT // 2
E_PAD = CPT * NW * B
RPT = NROWS // NS

_MESH = plsc.VectorSubcoreMesh(core_axis_name="c", subcore_axis_name="s")


def _worker_id():
    return lax.axis_index("s") * NC + lax.axis_index("c")


def _make_edge_pass(d_feat):
    @functools.partial(
        pl.kernel,
        mesh=_MESH,
        out_type=jax.ShapeDtypeStruct((NC * NROWS, d_feat), jnp.float32),
        scratch_types=[
            pltpu.VMEM((B,), jnp.int32),
            pltpu.VMEM((B,), jnp.int32),
            pltpu.VMEM((B,), jnp.int32),
            pltpu.VMEM((B,), jnp.int32),
            pltpu.VMEM((B, d_feat), jnp.float32),
            pltpu.VMEM((B, d_feat), jnp.float32),
            pltpu.VMEM_SHARED((NROWS, d_feat), jnp.float32),
            pltpu.SemaphoreType.DMA,
            pltpu.SemaphoreType.DMA,
        ],
    )
    def edge_pass(table_hbm, src_hbm, dst_hbm, zeros_hbm, out_hbm,
                  src0, src1, dst0, dst1, vals0, vals1, acc_sh, sem0, sem1):
        c = lax.axis_index("c")
        s = lax.axis_index("s")
        wid = _worker_id()
        r0 = s * RPT
        pltpu.sync_copy(zeros_hbm.at[pl.ds(r0, RPT)], acc_sh.at[pl.ds(r0, RPT)])
        plsc.subcore_barrier()

        ebase = wid * (CPT * B)

        pltpu.sync_copy(src_hbm.at[pl.ds(ebase, B)], src0)
        pltpu.sync_copy(dst_hbm.at[pl.ds(ebase, B)], dst0)
        pltpu.make_async_copy(table_hbm.at[src0], vals0, sem0).start()

        def body(g, carry):
            base1 = ebase + (2 * g + 1) * B
            pltpu.sync_copy(src_hbm.at[pl.ds(base1, B)], src1)
            pltpu.sync_copy(dst_hbm.at[pl.ds(base1, B)], dst1)
            pltpu.make_async_copy(table_hbm.at[src1], vals1, sem1).start()
            pltpu.make_async_copy(table_hbm.at[src0], vals0, sem0).wait()
            pltpu.sync_copy(vals0, acc_sh.at[dst0], add=True)

            @pl.when(g < CPT2 - 1)
            def _():
                base2 = ebase + (2 * g + 2) * B
                pltpu.sync_copy(src_hbm.at[pl.ds(base2, B)], src0)
                pltpu.sync_copy(dst_hbm.at[pl.ds(base2, B)], dst0)
                pltpu.make_async_copy(table_hbm.at[src0], vals0, sem0).start()

            pltpu.make_async_copy(table_hbm.at[src1], vals1, sem1).wait()
            pltpu.sync_copy(vals1, acc_sh.at[dst1], add=True)
            return carry

        lax.fori_loop(0, CPT2, body, 0)
        plsc.subcore_barrier()
        pltpu.sync_copy(acc_sh.at[pl.ds(r0, RPT)],
                        out_hbm.at[pl.ds(c * NROWS + r0, RPT)])

    return edge_pass


def _make_scalar_edge_pass():
    @functools.partial(
        pl.kernel,
        mesh=_MESH,
        out_type=jax.ShapeDtypeStruct((NC * NROWS,), jnp.float32),
        scratch_types=[
            pltpu.VMEM((B,), jnp.int32),
            pltpu.VMEM((B,), jnp.int32),
            pltpu.VMEM((B,), jnp.int32),
            pltpu.VMEM((B,), jnp.int32),
            pltpu.VMEM((B,), jnp.float32),
            pltpu.VMEM((B,), jnp.float32),
            pltpu.VMEM_SHARED((NROWS,), jnp.float32),
            pltpu.VMEM_SHARED((NROWS,), jnp.float32),
            pltpu.SemaphoreType.DMA,
            pltpu.SemaphoreType.DMA,
        ],
    )
    def scalar_pass(table_hbm, src_hbm, dst_hbm, zeros_hbm, out_hbm,
                    src0, src1, dst0, dst1, vals0, vals1, tab_sh, acc_sh,
                    sem0, sem1):
        c = lax.axis_index("c")
        s = lax.axis_index("s")
        wid = _worker_id()
        r0 = s * RPT
        pltpu.sync_copy(zeros_hbm.at[pl.ds(r0, RPT)], acc_sh.at[pl.ds(r0, RPT)])
        pltpu.sync_copy(table_hbm.at[pl.ds(r0, RPT)], tab_sh.at[pl.ds(r0, RPT)])
        plsc.subcore_barrier()

        ebase = wid * (CPT * B)

        pltpu.sync_copy(src_hbm.at[pl.ds(ebase, B)], src0)
        pltpu.sync_copy(dst_hbm.at[pl.ds(ebase, B)], dst0)
        pltpu.make_async_copy(tab_sh.at[src0], vals0, sem0).start()

        def body(g, carry):
            base1 = ebase + (2 * g + 1) * B
            pltpu.sync_copy(src_hbm.at[pl.ds(base1, B)], src1)
            pltpu.sync_copy(dst_hbm.at[pl.ds(base1, B)], dst1)
            pltpu.make_async_copy(tab_sh.at[src1], vals1, sem1).start()
            pltpu.make_async_copy(tab_sh.at[src0], vals0, sem0).wait()
            pltpu.sync_copy(vals0, acc_sh.at[dst0], add=True)

            @pl.when(g < CPT2 - 1)
            def _():
                base2 = ebase + (2 * g + 2) * B
                pltpu.sync_copy(src_hbm.at[pl.ds(base2, B)], src0)
                pltpu.sync_copy(dst_hbm.at[pl.ds(base2, B)], dst0)
                pltpu.make_async_copy(tab_sh.at[src0], vals0, sem0).start()

            pltpu.make_async_copy(tab_sh.at[src1], vals1, sem1).wait()
            pltpu.sync_copy(vals1, acc_sh.at[dst1], add=True)
            return carry

        lax.fori_loop(0, CPT2, body, 0)
        plsc.subcore_barrier()
        pltpu.sync_copy(acc_sh.at[pl.ds(r0, RPT)],
                        out_hbm.at[pl.ds(c * NROWS + r0, RPT)])

    return scalar_pass


def _make_deg_pass():
    @functools.partial(
        pl.kernel,
        mesh=_MESH,
        out_type=jax.ShapeDtypeStruct((NC * NROWS,), jnp.float32),
        scratch_types=[
            pltpu.VMEM((B,), jnp.int32),
            pltpu.VMEM((B,), jnp.int32),
            pltpu.VMEM((B,), jnp.float32),
            pltpu.VMEM_SHARED((NROWS,), jnp.float32),
            pltpu.SemaphoreType.DMA,
            pltpu.SemaphoreType.DMA,
        ],
    )
    def deg_pass(dst_hbm, ones_hbm, zeros_hbm, out_hbm, dst0, dst1, ones_v,
                 acc_sh, sem0, sem1):
        c = lax.axis_index("c")
        s = lax.axis_index("s")
        wid = _worker_id()
        r0 = s * RPT
        pltpu.sync_copy(zeros_hbm.at[pl.ds(r0, RPT)], acc_sh.at[pl.ds(r0, RPT)])
        pltpu.sync_copy(ones_hbm, ones_v)
        plsc.subcore_barrier()

        ebase = wid * (CPT * B)

        pltpu.make_async_copy(dst_hbm.at[pl.ds(ebase, B)], dst0, sem0).start()

        def body(g, carry):
            base1 = ebase + (2 * g + 1) * B
            pltpu.make_async_copy(dst_hbm.at[pl.ds(base1, B)], dst1, sem1).start()
            pltpu.make_async_copy(dst_hbm.at[pl.ds(ebase, B)], dst0, sem0).wait()
            pltpu.sync_copy(ones_v, acc_sh.at[dst0], add=True)

            @pl.when(g < CPT2 - 1)
            def _():
                base2 = ebase + (2 * g + 2) * B
                pltpu.make_async_copy(dst_hbm.at[pl.ds(base2, B)], dst0,
                                      sem0).start()

            pltpu.make_async_copy(dst_hbm.at[pl.ds(base1, B)], dst1, sem1).wait()
            pltpu.sync_copy(ones_v, acc_sh.at[dst1], add=True)
            return carry

        lax.fori_loop(0, CPT2, body, 0)
        plsc.subcore_barrier()
        pltpu.sync_copy(acc_sh.at[pl.ds(r0, RPT)],
                        out_hbm.at[pl.ds(c * NROWS + r0, RPT)])

    return deg_pass


_BR = 1024
_GRID = NROWS // _BR


def _tc_matmul_scale(x_pad, W1, degp):
    def body(x_ref, w_ref, degp_ref, xwp_ref, dis_ref):
        deg = degp_ref[0] + degp_ref[1] + 1.0
        dis = lax.rsqrt(deg)
        xw = jnp.dot(x_ref[...], w_ref[...], preferred_element_type=jnp.float32)
        xwp_ref[...] = dis * xw
        dis_ref[...] = dis

    return pl.pallas_call(
        body,
        grid=(_GRID,),
        in_specs=[
            pl.BlockSpec((_BR, D), lambda i: (i, 0)),
            pl.BlockSpec((D, D), lambda i: (0, 0)),
            pl.BlockSpec((NC, _BR, 1), lambda i: (0, i, 0)),
        ],
        out_specs=[
            pl.BlockSpec((_BR, D), lambda i: (i, 0)),
            pl.BlockSpec((_BR, 1), lambda i: (i, 0)),
        ],
        out_shape=[
            jax.ShapeDtypeStruct((NROWS, D), jnp.float32),
            jax.ShapeDtypeStruct((NROWS, 1), jnp.float32),
        ],
    )(x_pad, W1, degp)


def _tc_layer2(accp, xwp, dis, b1, W2, fc_W):
    def body(accp_ref, xwp_ref, dis_ref, b1_ref, w2_ref, fcw_ref, zp_ref):
        i = pl.program_id(0)
        acc = accp_ref[0] + accp_ref[1] + xwp_ref[...]
        h1 = jnp.maximum(dis_ref[...] * acc + b1_ref[...], 0.0)
        w2f = jnp.dot(w2_ref[...], fcw_ref[...], preferred_element_type=jnp.float32)
        z = jnp.dot(h1, w2f, preferred_element_type=jnp.float32)
        rows = i * _BR + lax.broadcasted_iota(jnp.int32, (_BR, 1), 0)
        zp_ref[...] = jnp.where(rows < N, dis_ref[...] * z, 0.0)

    return pl.pallas_call(
        body,
        grid=(_GRID,),
        in_specs=[
            pl.BlockSpec((NC, _BR, D), lambda i: (0, i, 0)),
            pl.BlockSpec((_BR, D), lambda i: (i, 0)),
            pl.BlockSpec((_BR, 1), lambda i: (i, 0)),
            pl.BlockSpec((1, D), lambda i: (0, 0)),
            pl.BlockSpec((D, 64), lambda i: (0, 0)),
            pl.BlockSpec((64, 1), lambda i: (0, 0)),
        ],
        out_specs=pl.BlockSpec((_BR, 1), lambda i: (i, 0)),
        out_shape=jax.ShapeDtypeStruct((NROWS, 1), jnp.float32),
    )(accp, xwp, dis, b1, W2, fc_W)


def _tc_final(acc2p, zp, dis, b2, fc_W, fc_b):
    def body(acc2p_ref, zp_ref, dis_ref, b2_ref, fcw_ref, fcb_ref, out_ref):
        t = dis_ref[...] * (acc2p_ref[0] + acc2p_ref[1] + zp_ref[...])
        bias = jnp.dot(b2_ref[...], fcw_ref[...],
                       preferred_element_type=jnp.float32) + fcb_ref[...]
        out_ref[...] = jax.nn.sigmoid(t + bias)

    return pl.pallas_call(
        body,
        grid=(_GRID,),
        in_specs=[
            pl.BlockSpec((NC, _BR, 1), lambda i: (0, i, 0)),
            pl.BlockSpec((_BR, 1), lambda i: (i, 0)),
            pl.BlockSpec((_BR, 1), lambda i: (i, 0)),
            pl.BlockSpec((1, 64), lambda i: (0, 0)),
            pl.BlockSpec((64, 1), lambda i: (0, 0)),
            pl.BlockSpec((1, 1), lambda i: (0, 0)),
        ],
        out_specs=pl.BlockSpec((_BR, 1), lambda i: (i, 0)),
        out_shape=jax.ShapeDtypeStruct((NROWS, 1), jnp.float32),
    )(acc2p, zp, dis, b2, fc_W, fc_b)


def kernel(x, edge_index, W1, b1, W2, b2, fc_W, fc_b):
    src = edge_index[0].astype(jnp.int32)
    dst = edge_index[1].astype(jnp.int32)
    npad = E_PAD - E
    pad_idx = N + (jnp.arange(npad, dtype=jnp.int32) % PAD_ROWS)
    src_p = jnp.concatenate([src, pad_idx])
    dst_p = jnp.concatenate([dst, pad_idx])

    zeros2d = jnp.zeros((NROWS, D), jnp.float32)
    zeros1d = jnp.zeros((NROWS,), jnp.float32)
    ones_b = jnp.ones((B,), jnp.float32)
    x_pad = jnp.concatenate([x.astype(jnp.float32),
                             jnp.zeros((NROWS - N, D), jnp.float32)])

    degp = _make_deg_pass()(dst_p, ones_b, zeros1d)
    degp = degp.reshape(NC, NROWS, 1)

    xwp, dis = _tc_matmul_scale(x_pad, W1, degp)

    accp = _make_edge_pass(D)(xwp, src_p, dst_p, zeros2d)
    accp = accp.reshape(NC, NROWS, D)

    zp = _tc_layer2(accp, xwp, dis, b1.reshape(1, D), W2, fc_W)

    acc2p = _make_scalar_edge_pass()(zp.reshape(NROWS), src_p, dst_p, zeros1d)
    acc2p = acc2p.reshape(NC, NROWS, 1)

    out = _tc_final(acc2p, zp, dis, b2.reshape(1, 64), fc_W,
                    fc_b.reshape(1, 1))
    return out[:N]


# speedup vs baseline: 40.8512x; 1.5237x over previous
import functools

import jax
import jax.numpy as jnp
from jax import lax
from jax.experimental import pallas as pl
from jax.experimental.pallas import tpu as pltpu
from jax.experimental.pallas import tpu_sc as plsc

N = 10000
E = 320000
D = 128
NROWS = 10240
PAD_ROWS = NROWS - N

NC = 2
NS = 16
NW = NC * NS
B = 128
KB = 8
CPT = 80
NG = CPT // KB
NG2 = NG // 2
E_PAD = CPT * NW * B
RPT = NROWS // NS

_MESH = plsc.VectorSubcoreMesh(core_axis_name="c", subcore_axis_name="s")


def _worker_id():
    return lax.axis_index("s") * NC + lax.axis_index("c")


def _gather_scatter_group(table, sb, db, acc_sh, vals, gsem, ssem):
    h_g = [None, None]
    h_s = [None, None]
    h_g[0] = pltpu.async_copy(table.at[sb.at[0]], vals[0], gsem[0])
    for k in range(KB):
        p = k % 2
        q = (k + 1) % 2
        if k < KB - 1:
            if k >= 1:
                h_s[q].wait()
            h_g[q] = pltpu.async_copy(table.at[sb.at[k + 1]], vals[q], gsem[q])
        h_g[p].wait()
        h_s[p] = pltpu.async_copy(vals[p], acc_sh.at[db.at[k]], ssem[p],
                                  add=True)
    h_s[0].wait()
    h_s[1].wait()


def _make_edge_pass(d_feat, stage_table):
    if d_feat == 1:
        tshape = (NROWS,)
        vshape = (B,)
    else:
        tshape = (NROWS, d_feat)
        vshape = (B, d_feat)

    scratch = [
        pltpu.VMEM((KB, B), jnp.int32),
        pltpu.VMEM((KB, B), jnp.int32),
        pltpu.VMEM((KB, B), jnp.int32),
        pltpu.VMEM((KB, B), jnp.int32),
        pltpu.VMEM(vshape, jnp.float32),
        pltpu.VMEM(vshape, jnp.float32),
        pltpu.VMEM_SHARED(tshape, jnp.float32),
    ] + ([pltpu.VMEM_SHARED(tshape, jnp.float32)] if stage_table else []) + [
        pltpu.SemaphoreType.DMA,
        pltpu.SemaphoreType.DMA,
        pltpu.SemaphoreType.DMA,
        pltpu.SemaphoreType.DMA,
        pltpu.SemaphoreType.DMA,
        pltpu.SemaphoreType.DMA,
    ]

    @functools.partial(
        pl.kernel,
        mesh=_MESH,
        out_type=jax.ShapeDtypeStruct((NC,) + tshape, jnp.float32),
        scratch_types=scratch,
    )
    def edge_pass(table_hbm, srcm_hbm, dstm_hbm, zeros_hbm, out_hbm,
                  sb0, sb1, db0, db1, valsA, valsB, acc_sh, *rest):
        if stage_table:
            tab_sh = rest[0]
            sems = rest[1:]
        else:
            tab_sh = None
            sems = rest
        semA, semB, ssemA, ssemB, semi0, semi1 = sems

        c = lax.axis_index("c")
        s = lax.axis_index("s")
        wid = _worker_id()
        r0 = s * RPT
        pltpu.sync_copy(zeros_hbm.at[pl.ds(r0, RPT)], acc_sh.at[pl.ds(r0, RPT)])
        if stage_table:
            pltpu.sync_copy(table_hbm.at[pl.ds(r0, RPT)],
                            tab_sh.at[pl.ds(r0, RPT)])
            table = tab_sh
        else:
            table = table_hbm
        plsc.subcore_barrier()

        row0 = wid * CPT
        vals = (valsA, valsB)
        gsem = (semA, semB)
        ssem = (ssemA, ssemB)

        pltpu.async_copy(srcm_hbm.at[pl.ds(row0, KB)], sb0, semi0)
        pltpu.async_copy(dstm_hbm.at[pl.ds(row0, KB)], db0, semi0)

        def body(t, carry):
            grow1 = row0 + (2 * t + 1) * KB
            hi_s = pltpu.async_copy(srcm_hbm.at[pl.ds(grow1, KB)], sb1, semi1)
            hi_d = pltpu.async_copy(dstm_hbm.at[pl.ds(grow1, KB)], db1, semi1)
            pltpu.make_async_copy(srcm_hbm.at[pl.ds(row0, KB)], sb0, semi0).wait()
            pltpu.make_async_copy(dstm_hbm.at[pl.ds(row0, KB)], db0, semi0).wait()
            _gather_scatter_group(table, sb0, db0, acc_sh, vals, gsem, ssem)

            @pl.when(t < NG2 - 1)
            def _():
                grow2 = row0 + (2 * t + 2) * KB
                pltpu.async_copy(srcm_hbm.at[pl.ds(grow2, KB)], sb0, semi0)
                pltpu.async_copy(dstm_hbm.at[pl.ds(grow2, KB)], db0, semi0)

            hi_s.wait()
            hi_d.wait()
            _gather_scatter_group(table, sb1, db1, acc_sh, vals, gsem, ssem)
            return carry

        lax.fori_loop(0, NG2, body, 0)
        plsc.subcore_barrier()
        pltpu.sync_copy(acc_sh.at[pl.ds(r0, RPT)],
                        out_hbm.at[c].at[pl.ds(r0, RPT)])

    return edge_pass


def _make_deg_pass():
    @functools.partial(
        pl.kernel,
        mesh=_MESH,
        out_type=jax.ShapeDtypeStruct((NC, NROWS), jnp.float32),
        scratch_types=[
            pltpu.VMEM((KB, B), jnp.int32),
            pltpu.VMEM((KB, B), jnp.int32),
            pltpu.VMEM((B,), jnp.float32),
            pltpu.VMEM_SHARED((NROWS,), jnp.float32),
            pltpu.SemaphoreType.DMA,
            pltpu.SemaphoreType.DMA,
            pltpu.SemaphoreType.DMA,
        ],
    )
    def deg_pass(dstm_hbm, ones_hbm, zeros_hbm, out_hbm, db0, db1, ones_v,
                 acc_sh, ssem, semi0, semi1):
        c = lax.axis_index("c")
        s = lax.axis_index("s")
        wid = _worker_id()
        r0 = s * RPT
        pltpu.sync_copy(zeros_hbm.at[pl.ds(r0, RPT)], acc_sh.at[pl.ds(r0, RPT)])
        pltpu.sync_copy(ones_hbm, ones_v)
        plsc.subcore_barrier()

        row0 = wid * CPT

        def scatter_group(db):
            hs = [pltpu.async_copy(ones_v, acc_sh.at[db.at[k]], ssem, add=True)
                  for k in range(KB)]
            for h in hs:
                h.wait()

        pltpu.async_copy(dstm_hbm.at[pl.ds(row0, KB)], db0, semi0)

        def body(t, carry):
            grow1 = row0 + (2 * t + 1) * KB
            hi_d = pltpu.async_copy(dstm_hbm.at[pl.ds(grow1, KB)], db1, semi1)
            pltpu.make_async_copy(dstm_hbm.at[pl.ds(row0, KB)], db0, semi0).wait()
            scatter_group(db0)

            @pl.when(t < NG2 - 1)
            def _():
                grow2 = row0 + (2 * t + 2) * KB
                pltpu.async_copy(dstm_hbm.at[pl.ds(grow2, KB)], db0, semi0)

            hi_d.wait()
            scatter_group(db1)
            return carry

        lax.fori_loop(0, NG2, body, 0)
        plsc.subcore_barrier()
        pltpu.sync_copy(acc_sh.at[pl.ds(r0, RPT)],
                        out_hbm.at[c].at[pl.ds(r0, RPT)])

    return deg_pass


_BR = 1024
_GRID = NROWS // _BR


def _tc_matmul_scale(x_pad, W1, degp):
    def body(x_ref, w_ref, degp_ref, xwp_ref, dis_ref):
        deg = degp_ref[0] + degp_ref[1] + 1.0
        dis = lax.rsqrt(deg)
        xw = jnp.dot(x_ref[...], w_ref[...], preferred_element_type=jnp.float32)
        xwp_ref[...] = dis * xw
        dis_ref[...] = dis

    return pl.pallas_call(
        body,
        grid=(_GRID,),
        in_specs=[
            pl.BlockSpec((_BR, D), lambda i: (i, 0)),
            pl.BlockSpec((D, D), lambda i: (0, 0)),
            pl.BlockSpec((NC, _BR, 1), lambda i: (0, i, 0)),
        ],
        out_specs=[
            pl.BlockSpec((_BR, D), lambda i: (i, 0)),
            pl.BlockSpec((_BR, 1), lambda i: (i, 0)),
        ],
        out_shape=[
            jax.ShapeDtypeStruct((NROWS, D), jnp.float32),
            jax.ShapeDtypeStruct((NROWS, 1), jnp.float32),
        ],
    )(x_pad, W1, degp)


def _tc_layer2(accp, xwp, dis, b1, W2, fc_W):
    def body(accp_ref, xwp_ref, dis_ref, b1_ref, w2_ref, fcw_ref, zp_ref):
        i = pl.program_id(0)
        acc = accp_ref[0] + accp_ref[1] + xwp_ref[...]
        h1 = jnp.maximum(dis_ref[...] * acc + b1_ref[...], 0.0)
        w2f = jnp.dot(w2_ref[...], fcw_ref[...], preferred_element_type=jnp.float32)
        z = jnp.dot(h1, w2f, preferred_element_type=jnp.float32)
        rows = i * _BR + lax.broadcasted_iota(jnp.int32, (_BR, 1), 0)
        zp_ref[...] = jnp.where(rows < N, dis_ref[...] * z, 0.0)

    return pl.pallas_call(
        body,
        grid=(_GRID,),
        in_specs=[
            pl.BlockSpec((NC, _BR, D), lambda i: (0, i, 0)),
            pl.BlockSpec((_BR, D), lambda i: (i, 0)),
            pl.BlockSpec((_BR, 1), lambda i: (i, 0)),
            pl.BlockSpec((1, D), lambda i: (0, 0)),
            pl.BlockSpec((D, 64), lambda i: (0, 0)),
            pl.BlockSpec((64, 1), lambda i: (0, 0)),
        ],
        out_specs=pl.BlockSpec((_BR, 1), lambda i: (i, 0)),
        out_shape=jax.ShapeDtypeStruct((NROWS, 1), jnp.float32),
    )(accp, xwp, dis, b1, W2, fc_W)


def _tc_final(acc2p, zp, dis, b2, fc_W, fc_b):
    def body(acc2p_ref, zp_ref, dis_ref, b2_ref, fcw_ref, fcb_ref, out_ref):
        t = dis_ref[...] * (acc2p_ref[0] + acc2p_ref[1] + zp_ref[...])
        bias = jnp.dot(b2_ref[...], fcw_ref[...],
                       preferred_element_type=jnp.float32) + fcb_ref[...]
        out_ref[...] = jax.nn.sigmoid(t + bias)

    return pl.pallas_call(
        body,
        grid=(_GRID,),
        in_specs=[
            pl.BlockSpec((NC, _BR, 1), lambda i: (0, i, 0)),
            pl.BlockSpec((_BR, 1), lambda i: (i, 0)),
            pl.BlockSpec((_BR, 1), lambda i: (i, 0)),
            pl.BlockSpec((1, 64), lambda i: (0, 0)),
            pl.BlockSpec((64, 1), lambda i: (0, 0)),
            pl.BlockSpec((1, 1), lambda i: (0, 0)),
        ],
        out_specs=pl.BlockSpec((_BR, 1), lambda i: (i, 0)),
        out_shape=jax.ShapeDtypeStruct((NROWS, 1), jnp.float32),
    )(acc2p, zp, dis, b2, fc_W, fc_b)


def kernel(x, edge_index, W1, b1, W2, b2, fc_W, fc_b):
    src = edge_index[0].astype(jnp.int32)
    dst = edge_index[1].astype(jnp.int32)
    npad = E_PAD - E
    pad_idx = N + (jnp.arange(npad, dtype=jnp.int32) % PAD_ROWS)
    srcm = jnp.concatenate([src, pad_idx]).reshape(NW * CPT, B)
    dstm = jnp.concatenate([dst, pad_idx]).reshape(NW * CPT, B)

    zeros2d = jnp.zeros((NROWS, D), jnp.float32)
    zeros1d = jnp.zeros((NROWS,), jnp.float32)
    ones_b = jnp.ones((B,), jnp.float32)
    x_pad = jnp.concatenate([x.astype(jnp.float32),
                             jnp.zeros((NROWS - N, D), jnp.float32)])

    degp = _make_deg_pass()(dstm, ones_b, zeros1d)
    degp = degp.reshape(NC, NROWS, 1)

    xwp, dis = _tc_matmul_scale(x_pad, W1, degp)

    accp = _make_edge_pass(D, stage_table=False)(xwp, srcm, dstm, zeros2d)

    zp = _tc_layer2(accp, xwp, dis, b1.reshape(1, D), W2, fc_W)

    acc2p = _make_edge_pass(1, stage_table=True)(zp.reshape(NROWS), srcm,
                                                 dstm, zeros1d)
    acc2p = acc2p.reshape(NC, NROWS, 1)

    out = _tc_final(acc2p, zp, dis, b2.reshape(1, 64), fc_W,
                    fc_b.reshape(1, 1))
    return out[:N]


# speedup vs baseline: 41.1878x; 1.0082x over previous
import functools

import jax
import jax.numpy as jnp
from jax import lax
from jax.experimental import pallas as pl
from jax.experimental.pallas import tpu as pltpu
from jax.experimental.pallas import tpu_sc as plsc

N = 10000
E = 320000
D = 128
NROWS = 10240
PAD_ROWS = NROWS - N

NC = 2
NS = 16
NW = NC * NS
B = 128
KB = 8
CPT = 80
NG = CPT // KB
NG2 = NG // 2
E_PAD = CPT * NW * B
RPT = NROWS // NS

_MESH = plsc.VectorSubcoreMesh(core_axis_name="c", subcore_axis_name="s")


def _worker_id():
    return lax.axis_index("s") * NC + lax.axis_index("c")


def _gather_scatter_group(table, sb, db, acc_sh, vals, gsem, ssem):
    nbuf = len(vals)
    h_g = [None] * nbuf
    h_s = [None] * nbuf
    state = {"started": 0}

    def start_one():
        j = state["started"]
        p = j % nbuf
        if h_s[p] is not None:
            h_s[p].wait()
        h_g[p] = pltpu.async_copy(table.at[sb.at[j]], vals[p], gsem[p])
        state["started"] = j + 1

    for _ in range(min(nbuf, KB)):
        start_one()
    for k in range(KB):
        p = k % nbuf
        h_g[p].wait()
        h_s[p] = pltpu.async_copy(vals[p], acc_sh.at[db.at[k]], ssem[p],
                                  add=True)
        if state["started"] < KB:
            start_one()
    for p in range(nbuf):
        if h_s[p] is not None:
            h_s[p].wait()


def _make_edge_pass(d_feat, stage_table, nbuf):
    if d_feat == 1:
        tshape = (NROWS,)
        vshape = (B,)
    else:
        tshape = (NROWS, d_feat)
        vshape = (B, d_feat)

    scratch = (
        [pltpu.VMEM((KB, B), jnp.int32)] * 4
        + [pltpu.VMEM(vshape, jnp.float32)] * nbuf
        + [pltpu.VMEM_SHARED(tshape, jnp.float32)]
        + ([pltpu.VMEM_SHARED(tshape, jnp.float32)] if stage_table else [])
        + [pltpu.SemaphoreType.DMA] * (2 * nbuf + 2)
    )

    @functools.partial(
        pl.kernel,
        mesh=_MESH,
        out_type=jax.ShapeDtypeStruct((NC,) + tshape, jnp.float32),
        scratch_types=scratch,
    )
    def edge_pass(table_hbm, srcm_hbm, dstm_hbm, zeros_hbm, out_hbm,
                  sb0, sb1, db0, db1, *rest):
        vals = tuple(rest[:nbuf])
        acc_sh = rest[nbuf]
        rest = rest[nbuf + 1:]
        if stage_table:
            tab_sh = rest[0]
            rest = rest[1:]
        else:
            tab_sh = None
        gsem = tuple(rest[:nbuf])
        ssem = tuple(rest[nbuf:2 * nbuf])
        semi0, semi1 = rest[2 * nbuf:]

        c = lax.axis_index("c")
        s = lax.axis_index("s")
        wid = _worker_id()
        r0 = s * RPT
        pltpu.sync_copy(zeros_hbm.at[pl.ds(r0, RPT)], acc_sh.at[pl.ds(r0, RPT)])
        if stage_table:
            pltpu.sync_copy(table_hbm.at[pl.ds(r0, RPT)],
                            tab_sh.at[pl.ds(r0, RPT)])
            table = tab_sh
        else:
            table = table_hbm
        plsc.subcore_barrier()

        row0 = wid * CPT

        pltpu.async_copy(srcm_hbm.at[pl.ds(row0, KB)], sb0, semi0)
        pltpu.async_copy(dstm_hbm.at[pl.ds(row0, KB)], db0, semi0)

        def body(t, carry):
            grow1 = row0 + (2 * t + 1) * KB
            hi_s = pltpu.async_copy(srcm_hbm.at[pl.ds(grow1, KB)], sb1, semi1)
            hi_d = pltpu.async_copy(dstm_hbm.at[pl.ds(grow1, KB)], db1, semi1)
            pltpu.make_async_copy(srcm_hbm.at[pl.ds(row0, KB)], sb0, semi0).wait()
            pltpu.make_async_copy(dstm_hbm.at[pl.ds(row0, KB)], db0, semi0).wait()
            _gather_scatter_group(table, sb0, db0, acc_sh, vals, gsem, ssem)

            @pl.when(t < NG2 - 1)
            def _():
                grow2 = row0 + (2 * t + 2) * KB
                pltpu.async_copy(srcm_hbm.at[pl.ds(grow2, KB)], sb0, semi0)
                pltpu.async_copy(dstm_hbm.at[pl.ds(grow2, KB)], db0, semi0)

            hi_s.wait()
            hi_d.wait()
            _gather_scatter_group(table, sb1, db1, acc_sh, vals, gsem, ssem)
            return carry

        lax.fori_loop(0, NG2, body, 0)
        plsc.subcore_barrier()
        pltpu.sync_copy(acc_sh.at[pl.ds(r0, RPT)],
                        out_hbm.at[c].at[pl.ds(r0, RPT)])

    return edge_pass


def _make_deg_pass():
    @functools.partial(
        pl.kernel,
        mesh=_MESH,
        out_type=jax.ShapeDtypeStruct((NC, NROWS), jnp.float32),
        scratch_types=[
            pltpu.VMEM((KB, B), jnp.int32),
            pltpu.VMEM((KB, B), jnp.int32),
            pltpu.VMEM((B,), jnp.float32),
            pltpu.VMEM_SHARED((NROWS,), jnp.float32),
            pltpu.SemaphoreType.DMA,
            pltpu.SemaphoreType.DMA,
            pltpu.SemaphoreType.DMA,
        ],
    )
    def deg_pass(dstm_hbm, ones_hbm, zeros_hbm, out_hbm, db0, db1, ones_v,
                 acc_sh, ssem, semi0, semi1):
        c = lax.axis_index("c")
        s = lax.axis_index("s")
        wid = _worker_id()
        r0 = s * RPT
        pltpu.sync_copy(zeros_hbm.at[pl.ds(r0, RPT)], acc_sh.at[pl.ds(r0, RPT)])
        pltpu.sync_copy(ones_hbm, ones_v)
        plsc.subcore_barrier()

        row0 = wid * CPT

        def scatter_group(db):
            hs = [pltpu.async_copy(ones_v, acc_sh.at[db.at[k]], ssem, add=True)
                  for k in range(KB)]
            for h in hs:
                h.wait()

        pltpu.async_copy(dstm_hbm.at[pl.ds(row0, KB)], db0, semi0)

        def body(t, carry):
            grow1 = row0 + (2 * t + 1) * KB
            hi_d = pltpu.async_copy(dstm_hbm.at[pl.ds(grow1, KB)], db1, semi1)
            pltpu.make_async_copy(dstm_hbm.at[pl.ds(row0, KB)], db0, semi0).wait()
            scatter_group(db0)

            @pl.when(t < NG2 - 1)
            def _():
                grow2 = row0 + (2 * t + 2) * KB
                pltpu.async_copy(dstm_hbm.at[pl.ds(grow2, KB)], db0, semi0)

            hi_d.wait()
            scatter_group(db1)
            return carry

        lax.fori_loop(0, NG2, body, 0)
        plsc.subcore_barrier()
        pltpu.sync_copy(acc_sh.at[pl.ds(r0, RPT)],
                        out_hbm.at[c].at[pl.ds(r0, RPT)])

    return deg_pass


_BR = 1024
_GRID = NROWS // _BR


def _tc_matmul_scale(x_pad, W1, degp):
    def body(x_ref, w_ref, degp_ref, xwp_ref, dis_ref):
        deg = degp_ref[0] + degp_ref[1] + 1.0
        dis = lax.rsqrt(deg)
        xw = jnp.dot(x_ref[...], w_ref[...], preferred_element_type=jnp.float32)
        xwp_ref[...] = dis * xw
        dis_ref[...] = dis

    return pl.pallas_call(
        body,
        grid=(_GRID,),
        in_specs=[
            pl.BlockSpec((_BR, D), lambda i: (i, 0)),
            pl.BlockSpec((D, D), lambda i: (0, 0)),
            pl.BlockSpec((NC, _BR, 1), lambda i: (0, i, 0)),
        ],
        out_specs=[
            pl.BlockSpec((_BR, D), lambda i: (i, 0)),
            pl.BlockSpec((_BR, 1), lambda i: (i, 0)),
        ],
        out_shape=[
            jax.ShapeDtypeStruct((NROWS, D), jnp.float32),
            jax.ShapeDtypeStruct((NROWS, 1), jnp.float32),
        ],
    )(x_pad, W1, degp)


def _tc_layer2(accp, xwp, dis, b1, W2, fc_W):
    def body(accp_ref, xwp_ref, dis_ref, b1_ref, w2_ref, fcw_ref, zp_ref):
        i = pl.program_id(0)
        acc = accp_ref[0] + accp_ref[1] + xwp_ref[...]
        h1 = jnp.maximum(dis_ref[...] * acc + b1_ref[...], 0.0)
        w2f = jnp.dot(w2_ref[...], fcw_ref[...], preferred_element_type=jnp.float32)
        z = jnp.dot(h1, w2f, preferred_element_type=jnp.float32)
        rows = i * _BR + lax.broadcasted_iota(jnp.int32, (_BR, 1), 0)
        zp_ref[...] = jnp.where(rows < N, dis_ref[...] * z, 0.0)

    return pl.pallas_call(
        body,
        grid=(_GRID,),
        in_specs=[
            pl.BlockSpec((NC, _BR, D), lambda i: (0, i, 0)),
            pl.BlockSpec((_BR, D), lambda i: (i, 0)),
            pl.BlockSpec((_BR, 1), lambda i: (i, 0)),
            pl.BlockSpec((1, D), lambda i: (0, 0)),
            pl.BlockSpec((D, 64), lambda i: (0, 0)),
            pl.BlockSpec((64, 1), lambda i: (0, 0)),
        ],
        out_specs=pl.BlockSpec((_BR, 1), lambda i: (i, 0)),
        out_shape=jax.ShapeDtypeStruct((NROWS, 1), jnp.float32),
    )(accp, xwp, dis, b1, W2, fc_W)


def _tc_final(acc2p, zp, dis, b2, fc_W, fc_b):
    def body(acc2p_ref, zp_ref, dis_ref, b2_ref, fcw_ref, fcb_ref, out_ref):
        t = dis_ref[...] * (acc2p_ref[0] + acc2p_ref[1] + zp_ref[...])
        bias = jnp.dot(b2_ref[...], fcw_ref[...],
                       preferred_element_type=jnp.float32) + fcb_ref[...]
        out_ref[...] = jax.nn.sigmoid(t + bias)

    return pl.pallas_call(
        body,
        grid=(_GRID,),
        in_specs=[
            pl.BlockSpec((NC, _BR, 1), lambda i: (0, i, 0)),
            pl.BlockSpec((_BR, 1), lambda i: (i, 0)),
            pl.BlockSpec((_BR, 1), lambda i: (i, 0)),
            pl.BlockSpec((1, 64), lambda i: (0, 0)),
            pl.BlockSpec((64, 1), lambda i: (0, 0)),
            pl.BlockSpec((1, 1), lambda i: (0, 0)),
        ],
        out_specs=pl.BlockSpec((_BR, 1), lambda i: (i, 0)),
        out_shape=jax.ShapeDtypeStruct((NROWS, 1), jnp.float32),
    )(acc2p, zp, dis, b2, fc_W, fc_b)


def kernel(x, edge_index, W1, b1, W2, b2, fc_W, fc_b):
    src = edge_index[0].astype(jnp.int32)
    dst = edge_index[1].astype(jnp.int32)
    npad = E_PAD - E
    pad_idx = N + (jnp.arange(npad, dtype=jnp.int32) % PAD_ROWS)
    srcm = jnp.concatenate([src, pad_idx]).reshape(NW * CPT, B)
    dstm = jnp.concatenate([dst, pad_idx]).reshape(NW * CPT, B)

    zeros2d = jnp.zeros((NROWS, D), jnp.float32)
    zeros1d = jnp.zeros((NROWS,), jnp.float32)
    ones_b = jnp.ones((B,), jnp.float32)
    x_pad = jnp.concatenate([x.astype(jnp.float32),
                             jnp.zeros((NROWS - N, D), jnp.float32)])

    degp = _make_deg_pass()(dstm, ones_b, zeros1d)
    degp = degp.reshape(NC, NROWS, 1)

    xwp, dis = _tc_matmul_scale(x_pad, W1, degp)

    accp = _make_edge_pass(D, stage_table=False, nbuf=2)(xwp, srcm, dstm, zeros2d)

    zp = _tc_layer2(accp, xwp, dis, b1.reshape(1, D), W2, fc_W)

    acc2p = _make_edge_pass(1, stage_table=True, nbuf=4)(zp.reshape(NROWS), srcm,
                                                 dstm, zeros1d)
    acc2p = acc2p.reshape(NC, NROWS, 1)

    out = _tc_final(acc2p, zp, dis, b2.reshape(1, 64), fc_W,
                    fc_b.reshape(1, 1))
    return out[:N]
